# Initial kernel scaffold; baseline (speedup 1.0000x reference)
import jax
import jax.numpy as jnp
from jax.experimental import pallas as pl


def kernel(nodes_dict):
    raise NotImplementedError("write your pallas kernel here")


# speedup vs baseline: 74.1816x; 74.1816x over previous
import functools

import jax
import jax.numpy as jnp
from jax.experimental import pallas as pl
from jax.experimental.pallas import tpu as pltpu

_EPS = 1e-6
_RSQ = 0.25
_T0 = 0.25 * (1 + 2.0 ** -23)
_B = 128


def _nms_body(n_valid, nb, xs_ref, ys_ref, zs_ref, mask_ref, cnt_ref, cl_ref):
    b = pl.program_id(0)
    cx = xs_ref[pl.ds(b, 1), :]
    cy = ys_ref[pl.ds(b, 1), :]
    cz = zs_ref[pl.ds(b, 1), :]
    cxt = cx.reshape(_B, 1)
    cyt = cy.reshape(_B, 1)
    czt = cz.reshape(_B, 1)

    def close_mat(px, py, pz):
        dx = cxt - px + _EPS
        dy = cyt - py + _EPS
        dz = czt - pz + _EPS
        s = dx * dx + dy * dy + dz * dz
        return jnp.logical_or(s <= _RSQ, s == _T0)

    def prior_body(a, sup):
        px = xs_ref[pl.ds(a, 1), :]
        py = ys_ref[pl.ds(a, 1), :]
        pz = zs_ref[pl.ds(a, 1), :]
        m = mask_ref[pl.ds(a, 1), :]
        hit = jnp.where(close_mat(px, py, pz), m, 0.0)
        return jnp.maximum(sup, jnp.max(hit, axis=1, keepdims=True))

    sup = jax.lax.fori_loop(
        0, b, prior_body, jnp.zeros((_B, 1), jnp.float32)
    )

    ri = jax.lax.broadcasted_iota(jnp.int32, (_B, _B), 0)
    ci = jax.lax.broadcasted_iota(jnp.int32, (_B, _B), 1)
    in_close = jnp.logical_and(close_mat(cx, cy, cz), ci < ri)
    cl_ref[:, :] = jnp.where(in_close, 1.0, 0.0)

    lane = jax.lax.broadcasted_iota(jnp.int32, (1, _B), 1)
    valid = (b * _B + lane) < n_valid
    presup = sup.reshape(1, _B)
    allowed = jnp.where(jnp.logical_and(valid, presup < 0.5), 1.0, 0.0)

    def seq_body(i, keep):
        row = cl_ref[pl.ds(i, 1), :]
        sup_i = jnp.max(row * keep)
        add = jnp.where(sup_i > 0.5, 0.0, 1.0) * allowed
        return jnp.where(lane == i, add, keep)

    keep = jax.lax.fori_loop(
        0, _B, seq_body, jnp.zeros((1, _B), jnp.float32)
    )
    mask_ref[pl.ds(b, 1), :] = keep

    @pl.when(b == nb - 1)
    def _():
        cnt_ref[:, :] = jnp.sum(mask_ref[:, :]).astype(jnp.int32).reshape(1, 1)


def kernel(nodes_dict):
    n = nodes_dict.shape[0]
    nb = (n + _B - 1) // _B
    npad = nb * _B
    nodes = jnp.pad(
        nodes_dict, ((0, npad - n), (0, 0)), constant_values=1e9
    ).astype(jnp.float32)
    xs = nodes[:, 0].reshape(nb, _B)
    ys = nodes[:, 1].reshape(nb, _B)
    zs = nodes[:, 2].reshape(nb, _B)

    mask_f, cnt = pl.pallas_call(
        functools.partial(_nms_body, n, nb),
        grid=(nb,),
        in_specs=[pl.BlockSpec((nb, _B), lambda b: (0, 0))] * 3,
        out_specs=[
            pl.BlockSpec((nb, _B), lambda b: (0, 0)),
            pl.BlockSpec((1, 1), lambda b: (0, 0)),
        ],
        out_shape=[
            jax.ShapeDtypeStruct((nb, _B), jnp.float32),
            jax.ShapeDtypeStruct((1, 1), jnp.int32),
        ],
        scratch_shapes=[pltpu.VMEM((_B, _B), jnp.float32)],
    )(xs, ys, zs)

    mask = mask_f.reshape(-1)[:n] > 0.5
    return (mask, cnt.reshape(1))


# speedup vs baseline: 554.4324x; 7.4740x over previous
import functools

import jax
import jax.numpy as jnp
from jax.experimental import pallas as pl
from jax.experimental.pallas import tpu as pltpu

_EPS = 1e-6
_RSQ = 0.25
_T0 = 0.25 * (1 + 2.0 ** -23)
_BIG = 1e30
_B = 128
_UNROLL = 4


def _nms_body(n_valid, nb, xs_ref, ys_ref, zs_ref, mask_ref, cnt_ref):
    b = pl.program_id(0)

    @pl.when(b == 0)
    def _():
        mask_ref[:, :] = jnp.zeros((nb, _B), jnp.float32)

    cx = xs_ref[pl.ds(b, 1), :]
    cy = ys_ref[pl.ds(b, 1), :]
    cz = zs_ref[pl.ds(b, 1), :]
    cxt = cx.reshape(_B, 1)
    cyt = cy.reshape(_B, 1)
    czt = cz.reshape(_B, 1)

    def sq_dist(px, py, pz):
        dx = cxt - px + _EPS
        dy = cyt - py + _EPS
        dz = czt - pz + _EPS
        return dx * dx + dy * dy + dz * dz

    def prior_body(a4, smin):
        base = a4 * _UNROLL
        x4 = xs_ref[pl.ds(base, _UNROLL), :]
        y4 = ys_ref[pl.ds(base, _UNROLL), :]
        z4 = zs_ref[pl.ds(base, _UNROLL), :]
        m4 = mask_ref[pl.ds(base, _UNROLL), :]
        for k in range(_UNROLL):
            s = sq_dist(x4[k : k + 1, :], y4[k : k + 1, :], z4[k : k + 1, :])
            masked = jnp.where(m4[k : k + 1, :] > 0.5, s, _BIG)
            smin = jnp.minimum(smin, masked)
        return smin

    nprior = (b + _UNROLL - 1) // _UNROLL
    smin = jax.lax.fori_loop(
        0, nprior, prior_body, jnp.full((_B, _B), _BIG, jnp.float32)
    )
    smin_col = jnp.min(smin, axis=1, keepdims=True)
    presup = jnp.logical_or(smin_col <= _RSQ, smin_col == _T0)

    s_in = sq_dist(cx, cy, cz)
    close_in = jnp.logical_or(s_in <= _RSQ, s_in == _T0)
    ri = jax.lax.broadcasted_iota(jnp.int32, (_B, _B), 0)
    ci = jax.lax.broadcasted_iota(jnp.int32, (_B, _B), 1)
    cl_low = jnp.where(
        jnp.logical_and(close_in, ci < ri), 1.0, 0.0
    )

    sub = jax.lax.broadcasted_iota(jnp.int32, (_B, 1), 0)
    valid = (b * _B + sub) < n_valid
    allowed = jnp.where(
        jnp.logical_and(valid, jnp.logical_not(presup)), 1.0, 0.0
    )

    def fp_cond(carry):
        _, changed = carry
        return changed

    def fp_body(carry):
        k, _ = carry
        hit = jnp.dot(cl_low, k, preferred_element_type=jnp.float32)
        newk = jnp.where(hit > 0.5, 0.0, allowed)
        return newk, jnp.any(newk != k)

    keep, _ = jax.lax.while_loop(
        fp_cond, fp_body, (allowed, True)
    )
    mask_ref[pl.ds(b, 1), :] = keep.reshape(1, _B)

    @pl.when(b == nb - 1)
    def _():
        cnt_ref[:, :] = jnp.sum(mask_ref[:, :]).astype(jnp.int32).reshape(1, 1)


def kernel(nodes_dict):
    n = nodes_dict.shape[0]
    nbu = _B * _UNROLL
    npad = ((n + nbu - 1) // nbu) * nbu
    nb = npad // _B
    nodes = jnp.pad(
        nodes_dict, ((0, npad - n), (0, 0)), constant_values=1e9
    ).astype(jnp.float32)
    xs = nodes[:, 0].reshape(nb, _B)
    ys = nodes[:, 1].reshape(nb, _B)
    zs = nodes[:, 2].reshape(nb, _B)

    mask_f, cnt = pl.pallas_call(
        functools.partial(_nms_body, n, nb),
        grid=(nb,),
        in_specs=[pl.BlockSpec((nb, _B), lambda b: (0, 0))] * 3,
        out_specs=[
            pl.BlockSpec((nb, _B), lambda b: (0, 0)),
            pl.BlockSpec((1, 1), lambda b: (0, 0)),
        ],
        out_shape=[
            jax.ShapeDtypeStruct((nb, _B), jnp.float32),
            jax.ShapeDtypeStruct((1, 1), jnp.int32),
        ],
    )(xs, ys, zs)

    mask = mask_f.reshape(-1)[:n] > 0.5
    return (mask, cnt.reshape(1))


# speedup vs baseline: 1269.4769x; 2.2897x over previous
import functools

import jax
import jax.numpy as jnp
from jax.experimental import pallas as pl
from jax.experimental.pallas import tpu as pltpu

_EPS = 1e-6
_RSQ = 0.25
_T0 = 0.25 * (1 + 2.0 ** -23)
_BIG = 1e30
_SENT = 1e9
_B = 128
_UNROLL = 4


def _nms_body(
    n_valid,
    nb,
    xs_ref,
    ys_ref,
    zs_ref,
    mask_ref,
    cnt_ref,
    kx_ref,
    ky_ref,
    kz_ref,
    kn_ref,
):
    b = pl.program_id(0)

    @pl.when(b == 0)
    def _():
        sent = jnp.full((nb, _B), _SENT, jnp.float32)
        kx_ref[:, :] = sent
        ky_ref[:, :] = sent
        kz_ref[:, :] = sent
        kn_ref[0] = 0

    cx = xs_ref[pl.ds(b, 1), :]
    cy = ys_ref[pl.ds(b, 1), :]
    cz = zs_ref[pl.ds(b, 1), :]
    cxt = cx.reshape(_B, 1)
    cyt = cy.reshape(_B, 1)
    czt = cz.reshape(_B, 1)

    def sq_dist(px, py, pz):
        dx = cxt - px + _EPS
        dy = cyt - py + _EPS
        dz = czt - pz + _EPS
        return dx * dx + dy * dy + dz * dz

    kcount = kn_ref[0]

    def prior_body(a4, smin):
        base = a4 * _UNROLL
        x4 = kx_ref[pl.ds(base, _UNROLL), :]
        y4 = ky_ref[pl.ds(base, _UNROLL), :]
        z4 = kz_ref[pl.ds(base, _UNROLL), :]
        for k in range(_UNROLL):
            s = sq_dist(x4[k : k + 1, :], y4[k : k + 1, :], z4[k : k + 1, :])
            smin = jnp.minimum(smin, s)
        return smin

    nprior = (kcount + _B * _UNROLL - 1) // (_B * _UNROLL)
    smin = jax.lax.fori_loop(
        0, nprior, prior_body, jnp.full((_B, _B), _BIG, jnp.float32)
    )
    smin_col = jnp.min(smin, axis=1, keepdims=True)
    presup = jnp.logical_or(smin_col <= _RSQ, smin_col == _T0)

    s_in = sq_dist(cx, cy, cz)
    close_in = jnp.logical_or(s_in <= _RSQ, s_in == _T0)
    ri = jax.lax.broadcasted_iota(jnp.int32, (_B, _B), 0)
    ci = jax.lax.broadcasted_iota(jnp.int32, (_B, _B), 1)
    cl_low = jnp.where(
        jnp.logical_and(close_in, ci < ri), 1.0, 0.0
    )

    sub = jax.lax.broadcasted_iota(jnp.int32, (_B, 1), 0)
    valid = (b * _B + sub) < n_valid
    allowed = jnp.where(
        jnp.logical_and(valid, jnp.logical_not(presup)), 1.0, 0.0
    )

    def fp_cond(carry):
        _, changed = carry
        return changed

    def fp_body(carry):
        k, _ = carry
        hit = jnp.dot(cl_low, k, preferred_element_type=jnp.float32)
        newk = jnp.where(hit > 0.5, 0.0, allowed)
        return newk, jnp.any(newk != k)

    keep, _ = jax.lax.while_loop(fp_cond, fp_body, (allowed, True))
    keep_row = keep.reshape(1, _B)
    mask_ref[pl.ds(b, 1), :] = keep_row

    ltri = jnp.where(ri >= ci, 1.0, 0.0)
    rank = jnp.dot(ltri, keep, preferred_element_type=jnp.float32)
    nkept = jnp.sum(keep).astype(jnp.int32)
    pos = kcount + rank.astype(jnp.int32) - 1
    row0 = kcount // _B

    def scatter_row(r):
        smat = jnp.where(
            jnp.logical_and(keep > 0.5, pos - r * _B == ci), 1.0, 0.0
        )
        hitm = jnp.dot(keep_row, smat, preferred_element_type=jnp.float32)
        hp = jax.lax.Precision.HIGHEST
        vx = jnp.dot(cx, smat, preferred_element_type=jnp.float32, precision=hp)
        vy = jnp.dot(cy, smat, preferred_element_type=jnp.float32, precision=hp)
        vz = jnp.dot(cz, smat, preferred_element_type=jnp.float32, precision=hp)
        wr = hitm > 0.5
        kx_ref[pl.ds(r, 1), :] = jnp.where(wr, vx, kx_ref[pl.ds(r, 1), :])
        ky_ref[pl.ds(r, 1), :] = jnp.where(wr, vy, ky_ref[pl.ds(r, 1), :])
        kz_ref[pl.ds(r, 1), :] = jnp.where(wr, vz, kz_ref[pl.ds(r, 1), :])

    scatter_row(row0)
    scatter_row(row0 + 1)
    kn_ref[0] = kcount + nkept

    @pl.when(b == nb - 1)
    def _():
        cnt_ref[:, :] = jnp.sum(mask_ref[:, :]).astype(jnp.int32).reshape(1, 1)


def kernel(nodes_dict):
    n = nodes_dict.shape[0]
    nbu = _B * _UNROLL
    npad = ((n + nbu - 1) // nbu) * nbu
    nb = npad // _B
    nodes = jnp.pad(
        nodes_dict, ((0, npad - n), (0, 0)), constant_values=_SENT
    ).astype(jnp.float32)
    xs = nodes[:, 0].reshape(nb, _B)
    ys = nodes[:, 1].reshape(nb, _B)
    zs = nodes[:, 2].reshape(nb, _B)

    mask_f, cnt = pl.pallas_call(
        functools.partial(_nms_body, n, nb),
        grid=(nb,),
        in_specs=[pl.BlockSpec((nb, _B), lambda b: (0, 0))] * 3,
        out_specs=[
            pl.BlockSpec((nb, _B), lambda b: (0, 0)),
            pl.BlockSpec((1, 1), lambda b: (0, 0)),
        ],
        out_shape=[
            jax.ShapeDtypeStruct((nb, _B), jnp.float32),
            jax.ShapeDtypeStruct((1, 1), jnp.int32),
        ],
        scratch_shapes=[
            pltpu.VMEM((nb, _B), jnp.float32),
            pltpu.VMEM((nb, _B), jnp.float32),
            pltpu.VMEM((nb, _B), jnp.float32),
            pltpu.SMEM((1,), jnp.int32),
        ],
    )(xs, ys, zs)

    mask = mask_f.reshape(-1)[:n] > 0.5
    return (mask, cnt.reshape(1))
